# baseline (speedup 1.0000x reference)
import jax
import jax.numpy as jnp
from jax import lax
from jax.experimental import pallas as pl
from jax.experimental.pallas import tpu as pltpu
from jax.experimental.pallas import tpu_sc as plsc

N_NODES = 10000
N_HYPEREDGES = 5000
NNZ = 320000
E_INNER = 640000
D = 128
NUM_CLASSES = 10
NUM_GRAPHS = 16
NUM_LAYERS = 2
INNER_LAYERS = 2

NT = 32
ROWS_PT = NNZ // NT
W = 512
NWIN = -(-ROWS_PT // W)
EB = 128
WIN = EB + 16


def _agg_body(x_hbm, srcs_hbm, dsts_hbm, tbl_hbm, out_hbm,
              outb, offv, widxA, wdstA, rowsA, widxB, wdstB, rowsB,
              semwA, semwB, semgA, semgB):
    cid = lax.axis_index("c")
    sid = lax.axis_index("s")
    wid = cid * 16 + sid
    iota = lax.iota(jnp.int32, 16)

    def fire_windows(b, eb_a, widx, wdst, semw):
        wa = eb_a + b * EB
        pltpu.async_copy(srcs_hbm.at[pl.ds(wa, WIN)], widx, semw)
        pltpu.async_copy(dsts_hbm.at[pl.ds(wa, WIN)], wdst, semw)

    def wait_windows(b, eb_a, widx, wdst, semw):
        wa = eb_a + b * EB
        pltpu.make_async_copy(srcs_hbm.at[pl.ds(wa, WIN)], widx, semw).wait()
        pltpu.make_async_copy(dsts_hbm.at[pl.ds(wa, WIN)], wdst, semw).wait()

    def fire_gather(widx, rows, semg):
        pltpu.async_copy(x_hbm.at[widx.at[pl.ds(0, EB)]],
                         rows.at[pl.ds(0, EB)], semg)
        pltpu.async_copy(x_hbm.at[widx.at[pl.ds(EB, 16)]],
                         rows.at[pl.ds(EB, 16)], semg)

    def wait_gather(widx, rows, semg):
        pltpu.make_async_copy(x_hbm.at[widx.at[pl.ds(0, EB)]],
                              rows.at[pl.ds(0, EB)], semg).wait()
        pltpu.make_async_copy(x_hbm.at[widx.at[pl.ds(EB, 16)]],
                              rows.at[pl.ds(EB, 16)], semg).wait()

    def rmw(b, eb, ee, eb_a, wr0, wdst, rows):
        wa = eb_a + b * EB
        bstart = eb + b * EB
        bend = jnp.minimum(bstart + EB, ee)

        def group(g, carry):
            pos = wa + g * 16 + iota
            dvv = wdst[pl.ds(g * 16, 16)]
            ok = (pos >= bstart) & (pos < bend)
            dvl = jnp.where(ok, dvv - wr0, W)
            for l in range(16):
                dv = dvl[l]
                e = g * 16 + l
                for k in range(D // 16):
                    outb[dv, pl.ds(k * 16, 16)] = (
                        outb[dv, pl.ds(k * 16, 16)]
                        + rows[e, pl.ds(k * 16, 16)])
            return carry

        lax.fori_loop(0, WIN // 16, group, 0)

    def win_body(w, carry):
        pltpu.sync_copy(tbl_hbm.at[wid * NWIN + w], offv)
        v = offv[...]
        eb = v[0]
        ee = v[1]
        wr0 = pl.multiple_of(v[2], 8)
        pltpu.sync_copy(x_hbm.at[pl.ds(wr0, W)], outb.at[pl.ds(0, W)])
        eb_a = pl.multiple_of(lax.div(eb, 8) * 8, 8)
        nb = lax.div(ee - eb + (EB - 1), EB)

        @pl.when(nb > 0)
        def _():
            fire_windows(0, eb_a, widxA, wdstA, semwA)
            wait_windows(0, eb_a, widxA, wdstA, semwA)
            fire_gather(widxA, rowsA, semgA)

            @pl.when(nb > 1)
            def _():
                fire_windows(1, eb_a, widxB, wdstB, semwB)

            def pair_body(i, c2):
                bA = 2 * i
                bB = 2 * i + 1

                @pl.when(bB < nb)
                def _():
                    wait_windows(bB, eb_a, widxB, wdstB, semwB)
                    fire_gather(widxB, rowsB, semgB)
                wait_gather(widxA, rowsA, semgA)
                rmw(bA, eb, ee, eb_a, wr0, wdstA, rowsA)

                @pl.when(bB + 1 < nb)
                def _():
                    fire_windows(bB + 1, eb_a, widxA, wdstA, semwA)

                @pl.when(bB < nb)
                def _():
                    wait_gather(widxB, rowsB, semgB)
                    rmw(bB, eb, ee, eb_a, wr0, wdstB, rowsB)

                @pl.when(bB + 1 < nb)
                def _():
                    wait_windows(bB + 1, eb_a, widxA, wdstA, semwA)
                    fire_gather(widxA, rowsA, semgA)

                @pl.when(bB + 2 < nb)
                def _():
                    fire_windows(bB + 2, eb_a, widxB, wdstB, semwB)
                return c2

            lax.fori_loop(0, lax.div(nb + 1, 2), pair_body, 0)

        pltpu.sync_copy(outb.at[pl.ds(0, W)], out_hbm.at[pl.ds(wr0, W)])
        return carry

    lax.fori_loop(0, NWIN, win_body, 0)


_sc_agg = pl.kernel(
    _agg_body,
    out_type=jax.ShapeDtypeStruct((NNZ, D), jnp.float32),
    mesh=plsc.VectorSubcoreMesh(core_axis_name="c", subcore_axis_name="s"),
    scratch_types=[
        pltpu.VMEM((W + 8, D), jnp.float32),
        pltpu.VMEM((16,), jnp.int32),
        pltpu.VMEM((WIN,), jnp.int32),
        pltpu.VMEM((WIN,), jnp.int32),
        pltpu.VMEM((WIN, D), jnp.float32),
        pltpu.VMEM((WIN,), jnp.int32),
        pltpu.VMEM((WIN,), jnp.int32),
        pltpu.VMEM((WIN, D), jnp.float32),
        pltpu.SemaphoreType.DMA,
        pltpu.SemaphoreType.DMA,
        pltpu.SemaphoreType.DMA,
        pltpu.SemaphoreType.DMA,
    ],
)


def _edge_prep(edge_index):
    src, dst = edge_index[0], edge_index[1]
    order = jnp.argsort(dst)
    src_s = jnp.take(src, order)
    dst_s = jnp.take(dst, order)
    wr0_rel = jnp.minimum(jnp.arange(NWIN, dtype=jnp.int32) * W, ROWS_PT - W)
    wr0 = (jnp.arange(NT, dtype=jnp.int32)[:, None] * ROWS_PT
           + wr0_rel[None, :]).reshape(-1)
    bnds = jnp.stack([wr0, wr0 + W], 1).reshape(-1)
    ebee = jnp.searchsorted(dst_s, bnds).astype(jnp.int32).reshape(-1, 2)
    tbl = jnp.concatenate(
        [ebee, wr0[:, None], jnp.zeros((NT * NWIN, 1), jnp.int32)], axis=1)
    tbl16 = jnp.tile(tbl, (1, 4))
    src_p = jnp.concatenate([src_s, jnp.zeros((2 * WIN + 8,), jnp.int32)])
    dst_p = jnp.concatenate([dst_s, jnp.full((2 * WIN + 8,), NNZ, jnp.int32)])
    return src_p, dst_p, tbl16


def _make_segsum(S, SW):
    def body(x_hbm, segp_hbm, tbl_hbm, zeros_hbm, out_hbm,
             outb, offv, wsegA, rowsA, wsegB, rowsB, semA, semB):
        cid = lax.axis_index("c")
        sid = lax.axis_index("s")
        wid = cid * 16 + sid
        iota = lax.iota(jnp.int32, 16)
        pltpu.sync_copy(tbl_hbm.at[wid], offv)
        v = offv[...]
        eb = v[0]
        ee = v[1]
        r0 = pl.multiple_of(v[2], 8)
        pltpu.sync_copy(zeros_hbm, outb.at[pl.ds(0, SW)])
        eb_a = pl.multiple_of(lax.div(eb, 8) * 8, 8)
        nb = lax.div(ee - eb + (EB - 1), EB)

        def wa_of(b):
            return pl.multiple_of(
                jnp.minimum(eb_a + b * EB, NNZ - WIN), 8)

        def fire(b, wseg, rows, sem):
            wa = wa_of(b)
            pltpu.async_copy(x_hbm.at[pl.ds(wa, WIN)], rows, sem)
            pltpu.async_copy(segp_hbm.at[pl.ds(wa, WIN)], wseg, sem)

        def wait(b, wseg, rows, sem):
            wa = wa_of(b)
            pltpu.make_async_copy(x_hbm.at[pl.ds(wa, WIN)], rows, sem).wait()
            pltpu.make_async_copy(segp_hbm.at[pl.ds(wa, WIN)], wseg, sem).wait()

        def rmw(b, wseg, rows):
            wa = wa_of(b)
            bstart = eb + b * EB
            bend = jnp.minimum(bstart + EB, ee)

            def group(g, carry):
                pos = wa + g * 16 + iota
                dvv = wseg[pl.ds(g * 16, 16)]
                ok = (pos >= bstart) & (pos < bend)
                dvl = jnp.where(ok, dvv - r0, SW)
                for l in range(16):
                    dv = dvl[l]
                    e = g * 16 + l
                    for k in range(D // 16):
                        outb[dv, pl.ds(k * 16, 16)] = (
                            outb[dv, pl.ds(k * 16, 16)]
                            + rows[e, pl.ds(k * 16, 16)])
                return carry

            lax.fori_loop(0, WIN // 16, group, 0)

        @pl.when(nb > 0)
        def _():
            fire(0, wsegA, rowsA, semA)

            @pl.when(nb > 1)
            def _():
                fire(1, wsegB, rowsB, semB)

            def pair_body(i, c2):
                bA = 2 * i
                bB = 2 * i + 1
                wait(bA, wsegA, rowsA, semA)
                rmw(bA, wsegA, rowsA)

                @pl.when(bB + 1 < nb)
                def _():
                    fire(bB + 1, wsegA, rowsA, semA)

                @pl.when(bB < nb)
                def _():
                    wait(bB, wsegB, rowsB, semB)
                    rmw(bB, wsegB, rowsB)

                @pl.when(bB + 2 < nb)
                def _():
                    fire(bB + 2, wsegB, rowsB, semB)
                return c2

            lax.fori_loop(0, lax.div(nb + 1, 2), pair_body, 0)

        def relu_row(r, carry):
            for k in range(D // 16):
                outb[r, pl.ds(k * 16, 16)] = jnp.maximum(
                    outb[r, pl.ds(k * 16, 16)], 0.0)
            return carry

        lax.fori_loop(0, SW, relu_row, 0)
        pltpu.sync_copy(outb.at[pl.ds(0, SW)], out_hbm.at[pl.ds(r0, SW)])

    return pl.kernel(
        body,
        out_type=jax.ShapeDtypeStruct((S, D), jnp.float32),
        mesh=plsc.VectorSubcoreMesh(core_axis_name="c", subcore_axis_name="s"),
        scratch_types=[
            pltpu.VMEM((SW + 8, D), jnp.float32),
            pltpu.VMEM((16,), jnp.int32),
            pltpu.VMEM((WIN,), jnp.int32),
            pltpu.VMEM((WIN, D), jnp.float32),
            pltpu.VMEM((WIN,), jnp.int32),
            pltpu.VMEM((WIN, D), jnp.float32),
            pltpu.SemaphoreType.DMA,
            pltpu.SemaphoreType.DMA,
        ],
    )


_sc_segsum_edge = _make_segsum(N_HYPEREDGES, 160)
_sc_segsum_node = _make_segsum(N_NODES, 320)


def _seg_prep(seg, S, SW):
    r0 = jnp.minimum(jnp.arange(NT, dtype=jnp.int32) * SW, S - SW)
    bnds = jnp.stack([r0, r0 + SW], 1).reshape(-1)
    ebee = jnp.searchsorted(seg, bnds).astype(jnp.int32).reshape(-1, 2)
    tbl16 = jnp.tile(jnp.concatenate(
        [ebee, r0[:, None], jnp.zeros((NT, 1), jnp.int32)], axis=1), (1, 4))
    segp = jnp.concatenate([seg, jnp.full((2 * WIN + 8,), S, jnp.int32)])
    return segp, tbl16


def _mlp_body(h_ref, w1_ref, b1_ref, w2_ref, b2_ref, o_ref):
    h = jnp.maximum(jnp.dot(h_ref[...], w1_ref[...],
                            preferred_element_type=jnp.float32)
                    + b1_ref[...], 0.0)
    o_ref[...] = jnp.maximum(jnp.dot(h, w2_ref[...],
                                     preferred_element_type=jnp.float32)
                             + b2_ref[...], 0.0)


def _gin_mlp(h, w1, b1, w2, b2):
    n = h.shape[0]
    blk = 2000
    row_spec = pl.BlockSpec((blk, D), lambda i: (i, 0))
    w_spec = pl.BlockSpec((D, D), lambda i: (0, 0))
    b_spec = pl.BlockSpec((1, D), lambda i: (0, 0))
    return pl.pallas_call(
        _mlp_body,
        grid=(n // blk,),
        in_specs=[row_spec, w_spec, b_spec, w_spec, b_spec],
        out_specs=row_spec,
        out_shape=jax.ShapeDtypeStruct((n, D), jnp.float32),
    )(h, w1, b1.reshape(1, D), w2, b2.reshape(1, D))


def _emb_body(x_ref, w_ref, b_ref, o_ref):
    o_ref[...] = jnp.dot(x_ref[...], w_ref[...],
                         preferred_element_type=jnp.float32) + b_ref[...]


def _emb(x, w, b):
    n = x.shape[0]
    blk = 2000
    return pl.pallas_call(
        _emb_body,
        grid=(n // blk,),
        in_specs=[pl.BlockSpec((blk, D), lambda i: (i, 0)),
                  pl.BlockSpec((D, D), lambda i: (0, 0)),
                  pl.BlockSpec((1, D), lambda i: (0, 0))],
        out_specs=pl.BlockSpec((blk, D), lambda i: (i, 0)),
        out_shape=jax.ShapeDtypeStruct((n, D), jnp.float32),
    )(x, w, b.reshape(1, D))


def kernel(x_N, W_emb, b_emb, gin_W1, gin_b1, gin_W2, gin_b2, W_pred, b_pred,
           ori_node_idx, node2edge, ori_edge_idx, edge2node,
           edge_index_N, edge_index_E, batch):
    node_x = _emb(x_N, W_emb, b_emb)
    prep_N = _edge_prep(edge_index_N)
    prep_E = _edge_prep(edge_index_E)
    segp_n2e, tbl_n2e = _seg_prep(node2edge, N_HYPEREDGES, 160)
    segp_e2n, tbl_e2n = _seg_prep(edge2node, N_NODES, 320)
    zeros_e = jnp.zeros((160, D), jnp.float32)
    zeros_n = jnp.zeros((320, D), jnp.float32)
    xs = [node_x]
    for l in range(NUM_LAYERS):
        _nx = node_x[ori_node_idx]
        for c in range(INNER_LAYERS):
            idx = l * 4 + c
            h = _sc_agg(_nx, *prep_N)
            _nx = _gin_mlp(h, gin_W1[idx], gin_b1[idx],
                           gin_W2[idx], gin_b2[idx])
        edge_x = _sc_segsum_edge(_nx, segp_n2e, tbl_n2e, zeros_e)
        _ex = edge_x[ori_edge_idx]
        for c in range(INNER_LAYERS):
            idx = l * 4 + 2 + c
            h = _sc_agg(_ex, *prep_E)
            _ex = _gin_mlp(h, gin_W1[idx], gin_b1[idx],
                           gin_W2[idx], gin_b2[idx])
        node_x = _sc_segsum_node(_ex, segp_e2n, tbl_e2n, zeros_n)
        xs.append(node_x)
    score = jnp.zeros((NUM_GRAPHS, NUM_CLASSES), jnp.float32)
    for i, x in enumerate(xs):
        pooled = jax.ops.segment_sum(x[ori_node_idx], batch,
                                     num_segments=NUM_GRAPHS)
        score = score + pooled @ W_pred[i] + b_pred[i]
    return score


# speedup vs baseline: 1.0478x; 1.0478x over previous
import jax
import jax.numpy as jnp
from jax import lax
from jax.experimental import pallas as pl
from jax.experimental.pallas import tpu as pltpu
from jax.experimental.pallas import tpu_sc as plsc

N_NODES = 10000
N_HYPEREDGES = 5000
NNZ = 320000
E_INNER = 640000
D = 128
NUM_CLASSES = 10
NUM_GRAPHS = 16
NUM_LAYERS = 2
INNER_LAYERS = 2

NT = 32
ROWS_PT = NNZ // NT
W = 512
NWIN = -(-ROWS_PT // W)
EB = 128
WIN = EB + 16


def _agg_body(x_hbm, srcs_hbm, dsts_hbm, tbl_hbm, out_hbm,
              outb, offv, widxA, wdstA, rowsA, widxB, wdstB, rowsB,
              semwA, semwB, semgA, semgB):
    cid = lax.axis_index("c")
    sid = lax.axis_index("s")
    wid = cid * 16 + sid
    iota = lax.iota(jnp.int32, 16)

    def fire_windows(b, eb_a, widx, wdst, semw):
        wa = eb_a + b * EB
        pltpu.async_copy(srcs_hbm.at[pl.ds(wa, WIN)], widx, semw)
        pltpu.async_copy(dsts_hbm.at[pl.ds(wa, WIN)], wdst, semw)

    def wait_windows(b, eb_a, widx, wdst, semw):
        wa = eb_a + b * EB
        pltpu.make_async_copy(srcs_hbm.at[pl.ds(wa, WIN)], widx, semw).wait()
        pltpu.make_async_copy(dsts_hbm.at[pl.ds(wa, WIN)], wdst, semw).wait()

    def fire_gather(widx, rows, semg):
        pltpu.async_copy(x_hbm.at[widx.at[pl.ds(0, EB)]],
                         rows.at[pl.ds(0, EB)], semg)
        pltpu.async_copy(x_hbm.at[widx.at[pl.ds(EB, 16)]],
                         rows.at[pl.ds(EB, 16)], semg)

    def wait_gather(widx, rows, semg):
        pltpu.make_async_copy(x_hbm.at[widx.at[pl.ds(0, EB)]],
                              rows.at[pl.ds(0, EB)], semg).wait()
        pltpu.make_async_copy(x_hbm.at[widx.at[pl.ds(EB, 16)]],
                              rows.at[pl.ds(EB, 16)], semg).wait()

    def rmw(b, eb, ee, eb_a, wr0, wdst, rows):
        wa = eb_a + b * EB
        bstart = eb + b * EB
        bend = jnp.minimum(bstart + EB, ee)

        def group(g, carry):
            pos = wa + g * 16 + iota
            dvv = wdst[pl.ds(g * 16, 16)]
            ok = (pos >= bstart) & (pos < bend)
            dvl = jnp.where(ok, dvv - wr0, W)
            for l in range(16):
                dv = dvl[l]
                e = g * 16 + l
                for k in range(D // 16):
                    outb[dv, pl.ds(k * 16, 16)] = (
                        outb[dv, pl.ds(k * 16, 16)]
                        + rows[e, pl.ds(k * 16, 16)])
            return carry

        lax.fori_loop(0, WIN // 16, group, 0)

    def win_body(w, carry):
        pltpu.sync_copy(tbl_hbm.at[wid * NWIN + w], offv)
        v = offv[...]
        eb = v[0]
        ee = v[1]
        wr0 = pl.multiple_of(v[2], 8)
        pltpu.sync_copy(x_hbm.at[pl.ds(wr0, W)], outb.at[pl.ds(0, W)])
        eb_a = pl.multiple_of(lax.div(eb, 8) * 8, 8)
        nb = lax.div(ee - eb + (EB - 1), EB)

        @pl.when(nb > 0)
        def _():
            fire_windows(0, eb_a, widxA, wdstA, semwA)
            wait_windows(0, eb_a, widxA, wdstA, semwA)
            fire_gather(widxA, rowsA, semgA)

            @pl.when(nb > 1)
            def _():
                fire_windows(1, eb_a, widxB, wdstB, semwB)

            def pair_body(i, c2):
                bA = 2 * i
                bB = 2 * i + 1

                @pl.when(bB < nb)
                def _():
                    wait_windows(bB, eb_a, widxB, wdstB, semwB)
                    fire_gather(widxB, rowsB, semgB)
                wait_gather(widxA, rowsA, semgA)
                rmw(bA, eb, ee, eb_a, wr0, wdstA, rowsA)

                @pl.when(bB + 1 < nb)
                def _():
                    fire_windows(bB + 1, eb_a, widxA, wdstA, semwA)

                @pl.when(bB < nb)
                def _():
                    wait_gather(widxB, rowsB, semgB)
                    rmw(bB, eb, ee, eb_a, wr0, wdstB, rowsB)

                @pl.when(bB + 1 < nb)
                def _():
                    wait_windows(bB + 1, eb_a, widxA, wdstA, semwA)
                    fire_gather(widxA, rowsA, semgA)

                @pl.when(bB + 2 < nb)
                def _():
                    fire_windows(bB + 2, eb_a, widxB, wdstB, semwB)
                return c2

            lax.fori_loop(0, lax.div(nb + 1, 2), pair_body, 0)

        pltpu.sync_copy(outb.at[pl.ds(0, W)], out_hbm.at[pl.ds(wr0, W)])
        return carry

    lax.fori_loop(0, NWIN, win_body, 0)


_sc_agg = pl.kernel(
    _agg_body,
    out_type=jax.ShapeDtypeStruct((NNZ, D), jnp.float32),
    mesh=plsc.VectorSubcoreMesh(core_axis_name="c", subcore_axis_name="s"),
    scratch_types=[
        pltpu.VMEM((W + 8, D), jnp.float32),
        pltpu.VMEM((16,), jnp.int32),
        pltpu.VMEM((WIN,), jnp.int32),
        pltpu.VMEM((WIN,), jnp.int32),
        pltpu.VMEM((WIN, D), jnp.float32),
        pltpu.VMEM((WIN,), jnp.int32),
        pltpu.VMEM((WIN,), jnp.int32),
        pltpu.VMEM((WIN, D), jnp.float32),
        pltpu.SemaphoreType.DMA,
        pltpu.SemaphoreType.DMA,
        pltpu.SemaphoreType.DMA,
        pltpu.SemaphoreType.DMA,
    ],
)


def _edge_prep(edge_index):
    src, dst = edge_index[0], edge_index[1]
    order = jnp.argsort(dst)
    src_s = jnp.take(src, order)
    dst_s = jnp.take(dst, order)
    wr0_rel = jnp.minimum(jnp.arange(NWIN, dtype=jnp.int32) * W, ROWS_PT - W)
    wr0 = (jnp.arange(NT, dtype=jnp.int32)[:, None] * ROWS_PT
           + wr0_rel[None, :]).reshape(-1)
    bnds = jnp.stack([wr0, wr0 + W], 1).reshape(-1)
    ebee = jnp.searchsorted(dst_s, bnds).astype(jnp.int32).reshape(-1, 2)
    tbl = jnp.concatenate(
        [ebee, wr0[:, None], jnp.zeros((NT * NWIN, 1), jnp.int32)], axis=1)
    tbl16 = jnp.tile(tbl, (1, 4))
    src_p = jnp.concatenate([src_s, jnp.zeros((2 * WIN + 8,), jnp.int32)])
    dst_p = jnp.concatenate([dst_s, jnp.full((2 * WIN + 8,), NNZ, jnp.int32)])
    return src_p, dst_p, tbl16


def _make_segsum(S, SW):
    def body(x_hbm, segp_hbm, tbl_hbm, zeros_hbm, out_hbm,
             outb, offv, wsegA, rowsA, wsegB, rowsB, semA, semB):
        cid = lax.axis_index("c")
        sid = lax.axis_index("s")
        wid = cid * 16 + sid
        iota = lax.iota(jnp.int32, 16)
        pltpu.sync_copy(tbl_hbm.at[wid], offv)
        v = offv[...]
        eb = v[0]
        ee = v[1]
        r0 = pl.multiple_of(v[2], 8)
        pltpu.sync_copy(zeros_hbm, outb.at[pl.ds(0, SW)])
        eb_a = pl.multiple_of(lax.div(eb, 8) * 8, 8)
        nb = lax.div(ee - eb + (EB - 1), EB)

        def wa_of(b):
            return pl.multiple_of(
                jnp.minimum(eb_a + b * EB, NNZ - WIN), 8)

        def fire(b, wseg, rows, sem):
            wa = wa_of(b)
            pltpu.async_copy(x_hbm.at[pl.ds(wa, WIN)], rows, sem)
            pltpu.async_copy(segp_hbm.at[pl.ds(wa, WIN)], wseg, sem)

        def wait(b, wseg, rows, sem):
            wa = wa_of(b)
            pltpu.make_async_copy(x_hbm.at[pl.ds(wa, WIN)], rows, sem).wait()
            pltpu.make_async_copy(segp_hbm.at[pl.ds(wa, WIN)], wseg, sem).wait()

        def rmw(b, wseg, rows):
            wa = wa_of(b)
            bstart = eb + b * EB
            bend = jnp.minimum(bstart + EB, ee)

            def group(g, carry):
                pos = wa + g * 16 + iota
                dvv = wseg[pl.ds(g * 16, 16)]
                ok = (pos >= bstart) & (pos < bend)
                dvl = jnp.where(ok, dvv - r0, SW)
                for l in range(16):
                    dv = dvl[l]
                    e = g * 16 + l
                    for k in range(D // 16):
                        outb[dv, pl.ds(k * 16, 16)] = (
                            outb[dv, pl.ds(k * 16, 16)]
                            + rows[e, pl.ds(k * 16, 16)])
                return carry

            lax.fori_loop(0, WIN // 16, group, 0)

        @pl.when(nb > 0)
        def _():
            fire(0, wsegA, rowsA, semA)

            @pl.when(nb > 1)
            def _():
                fire(1, wsegB, rowsB, semB)

            def pair_body(i, c2):
                bA = 2 * i
                bB = 2 * i + 1
                wait(bA, wsegA, rowsA, semA)
                rmw(bA, wsegA, rowsA)

                @pl.when(bB + 1 < nb)
                def _():
                    fire(bB + 1, wsegA, rowsA, semA)

                @pl.when(bB < nb)
                def _():
                    wait(bB, wsegB, rowsB, semB)
                    rmw(bB, wsegB, rowsB)

                @pl.when(bB + 2 < nb)
                def _():
                    fire(bB + 2, wsegB, rowsB, semB)
                return c2

            lax.fori_loop(0, lax.div(nb + 1, 2), pair_body, 0)

        def relu_row(r, carry):
            for k in range(D // 16):
                outb[r, pl.ds(k * 16, 16)] = jnp.maximum(
                    outb[r, pl.ds(k * 16, 16)], 0.0)
            return carry

        lax.fori_loop(0, SW, relu_row, 0)
        pltpu.sync_copy(outb.at[pl.ds(0, SW)], out_hbm.at[pl.ds(r0, SW)])

    return pl.kernel(
        body,
        out_type=jax.ShapeDtypeStruct((S, D), jnp.float32),
        mesh=plsc.VectorSubcoreMesh(core_axis_name="c", subcore_axis_name="s"),
        scratch_types=[
            pltpu.VMEM((SW + 8, D), jnp.float32),
            pltpu.VMEM((16,), jnp.int32),
            pltpu.VMEM((WIN,), jnp.int32),
            pltpu.VMEM((WIN, D), jnp.float32),
            pltpu.VMEM((WIN,), jnp.int32),
            pltpu.VMEM((WIN, D), jnp.float32),
            pltpu.SemaphoreType.DMA,
            pltpu.SemaphoreType.DMA,
        ],
    )


_sc_segsum_edge = _make_segsum(N_HYPEREDGES, 160)
_sc_segsum_node = _make_segsum(N_NODES, 320)


def _seg_prep(seg, S, SW):
    r0 = jnp.minimum(jnp.arange(NT, dtype=jnp.int32) * SW, S - SW)
    bnds = jnp.stack([r0, r0 + SW], 1).reshape(-1)
    ebee = jnp.searchsorted(seg, bnds).astype(jnp.int32).reshape(-1, 2)
    tbl16 = jnp.tile(jnp.concatenate(
        [ebee, r0[:, None], jnp.zeros((NT, 1), jnp.int32)], axis=1), (1, 4))
    segp = jnp.concatenate([seg, jnp.full((2 * WIN + 8,), S, jnp.int32)])
    return segp, tbl16


def _pool_body(g_hbm, seg_hbm, out_hbm,
               pacc, wsA, rwA, wsB, rwB, wst, rwt, semA, semB):
    cid = lax.axis_index("c")
    sid = lax.axis_index("s")
    wid = cid * 16 + sid
    base = pl.multiple_of(wid * ROWS_PT, 8)
    NB = ROWS_PT // EB
    zero = jnp.zeros((16,), jnp.float32)
    for r in range(NUM_GRAPHS):
        for k in range(D // 16):
            pacc[r, pl.ds(k * 16, 16)] = zero

    def fire(b, ws, rw, sem):
        wa = pl.multiple_of(base + b * EB, 8)
        pltpu.async_copy(g_hbm.at[pl.ds(wa, EB)], rw, sem)
        pltpu.async_copy(seg_hbm.at[pl.ds(wa, EB)], ws, sem)

    def wait(b, ws, rw, sem):
        wa = pl.multiple_of(base + b * EB, 8)
        pltpu.make_async_copy(g_hbm.at[pl.ds(wa, EB)], rw, sem).wait()
        pltpu.make_async_copy(seg_hbm.at[pl.ds(wa, EB)], ws, sem).wait()

    def rmw(ws, rw, ng):
        def group(gi, c):
            dvl = ws[pl.ds(gi * 16, 16)]
            for l in range(16):
                dv = dvl[l]
                e = gi * 16 + l
                for k in range(D // 16):
                    pacc[dv, pl.ds(k * 16, 16)] = (
                        pacc[dv, pl.ds(k * 16, 16)]
                        + rw[e, pl.ds(k * 16, 16)])
            return c
        lax.fori_loop(0, ng, group, 0)

    fire(0, wsA, rwA, semA)
    fire(1, wsB, rwB, semB)

    def pair_body(i, c2):
        bA = 2 * i
        bB = 2 * i + 1
        wait(bA, wsA, rwA, semA)
        rmw(wsA, rwA, EB // 16)

        @pl.when(bB + 1 < NB)
        def _():
            fire(bB + 1, wsA, rwA, semA)
        wait(bB, wsB, rwB, semB)
        rmw(wsB, rwB, EB // 16)

        @pl.when(bB + 2 < NB)
        def _():
            fire(bB + 2, wsB, rwB, semB)
        return c2

    lax.fori_loop(0, NB // 2, pair_body, 0)
    ta = pl.multiple_of(base + NB * EB, 8)
    pltpu.sync_copy(g_hbm.at[pl.ds(ta, ROWS_PT - NB * EB)], rwt)
    pltpu.sync_copy(seg_hbm.at[pl.ds(ta, ROWS_PT - NB * EB)], wst)
    rmw(wst, rwt, (ROWS_PT - NB * EB) // 16)
    pltpu.sync_copy(pacc.at[pl.ds(0, NUM_GRAPHS)],
                    out_hbm.at[pl.ds(wid * NUM_GRAPHS, NUM_GRAPHS)])


_sc_pool = pl.kernel(
    _pool_body,
    out_type=jax.ShapeDtypeStruct((NT * NUM_GRAPHS, D), jnp.float32),
    mesh=plsc.VectorSubcoreMesh(core_axis_name="c", subcore_axis_name="s"),
    scratch_types=[
        pltpu.VMEM((NUM_GRAPHS, D), jnp.float32),
        pltpu.VMEM((EB,), jnp.int32),
        pltpu.VMEM((EB, D), jnp.float32),
        pltpu.VMEM((EB,), jnp.int32),
        pltpu.VMEM((EB, D), jnp.float32),
        pltpu.VMEM((16,), jnp.int32),
        pltpu.VMEM((16, D), jnp.float32),
        pltpu.SemaphoreType.DMA,
        pltpu.SemaphoreType.DMA,
    ],
)


def _mlp_body(h_ref, w1_ref, b1_ref, w2_ref, b2_ref, o_ref):
    h = jnp.maximum(jnp.dot(h_ref[...], w1_ref[...],
                            preferred_element_type=jnp.float32)
                    + b1_ref[...], 0.0)
    o_ref[...] = jnp.maximum(jnp.dot(h, w2_ref[...],
                                     preferred_element_type=jnp.float32)
                             + b2_ref[...], 0.0)


def _gin_mlp(h, w1, b1, w2, b2):
    n = h.shape[0]
    blk = 2000
    row_spec = pl.BlockSpec((blk, D), lambda i: (i, 0))
    w_spec = pl.BlockSpec((D, D), lambda i: (0, 0))
    b_spec = pl.BlockSpec((1, D), lambda i: (0, 0))
    return pl.pallas_call(
        _mlp_body,
        grid=(n // blk,),
        in_specs=[row_spec, w_spec, b_spec, w_spec, b_spec],
        out_specs=row_spec,
        out_shape=jax.ShapeDtypeStruct((n, D), jnp.float32),
    )(h, w1, b1.reshape(1, D), w2, b2.reshape(1, D))


def _emb_body(x_ref, w_ref, b_ref, o_ref):
    o_ref[...] = jnp.dot(x_ref[...], w_ref[...],
                         preferred_element_type=jnp.float32) + b_ref[...]


def _emb(x, w, b):
    n = x.shape[0]
    blk = 2000
    return pl.pallas_call(
        _emb_body,
        grid=(n // blk,),
        in_specs=[pl.BlockSpec((blk, D), lambda i: (i, 0)),
                  pl.BlockSpec((D, D), lambda i: (0, 0)),
                  pl.BlockSpec((1, D), lambda i: (0, 0))],
        out_specs=pl.BlockSpec((blk, D), lambda i: (i, 0)),
        out_shape=jax.ShapeDtypeStruct((n, D), jnp.float32),
    )(x, w, b.reshape(1, D))


def kernel(x_N, W_emb, b_emb, gin_W1, gin_b1, gin_W2, gin_b2, W_pred, b_pred,
           ori_node_idx, node2edge, ori_edge_idx, edge2node,
           edge_index_N, edge_index_E, batch):
    node_x = _emb(x_N, W_emb, b_emb)
    prep_N = _edge_prep(edge_index_N)
    prep_E = _edge_prep(edge_index_E)
    segp_n2e, tbl_n2e = _seg_prep(node2edge, N_HYPEREDGES, 160)
    segp_e2n, tbl_e2n = _seg_prep(edge2node, N_NODES, 320)
    zeros_e = jnp.zeros((160, D), jnp.float32)
    zeros_n = jnp.zeros((320, D), jnp.float32)
    xs = [node_x]
    for l in range(NUM_LAYERS):
        _nx = node_x[ori_node_idx]
        for c in range(INNER_LAYERS):
            idx = l * 4 + c
            h = _sc_agg(_nx, *prep_N)
            _nx = _gin_mlp(h, gin_W1[idx], gin_b1[idx],
                           gin_W2[idx], gin_b2[idx])
        edge_x = _sc_segsum_edge(_nx, segp_n2e, tbl_n2e, zeros_e)
        _ex = edge_x[ori_edge_idx]
        for c in range(INNER_LAYERS):
            idx = l * 4 + 2 + c
            h = _sc_agg(_ex, *prep_E)
            _ex = _gin_mlp(h, gin_W1[idx], gin_b1[idx],
                           gin_W2[idx], gin_b2[idx])
        node_x = _sc_segsum_node(_ex, segp_e2n, tbl_e2n, zeros_n)
        xs.append(node_x)
    score = jnp.zeros((NUM_GRAPHS, NUM_CLASSES), jnp.float32)
    for i, x in enumerate(xs):
        partials = _sc_pool(x[ori_node_idx], batch)
        pooled = partials.reshape(NT, NUM_GRAPHS, D).sum(axis=0)
        score = score + pooled @ W_pred[i] + b_pred[i]
    return score
